# baseline (speedup 1.0000x reference)
import functools

import jax
import jax.numpy as jnp
from jax import lax
from jax.experimental import pallas as pl
from jax.experimental.pallas import tpu as pltpu
from jax.experimental.pallas import tpu_sc as plsc

N = 256
E = 65280
EPAD = 65536
IN = 7
HID = 128
HEADS = 2
OUT = N * 6

NC = 2
NS = 16
EPT = EPAD // (NC * NS)
ROWS = EPT // 128
SLC = EPAD // NS

FCS = 4
FCK = N // FCS


def _count_body(src_hbm, dst_hbm, zero_hbm, out_hbm,
                s_v, d_v, idx2, ones_v, cnt_sh, sem):
    cid = lax.axis_index("c")
    sid = lax.axis_index("s")
    tid = cid * NS + sid
    base = tid * EPT

    pltpu.sync_copy(src_hbm.at[pl.ds(base, EPT)], s_v)
    pltpu.sync_copy(dst_hbm.at[pl.ds(base, EPT)], d_v)
    pltpu.sync_copy(zero_hbm.at[pl.ds(sid * SLC, SLC)], cnt_sh.at[pl.ds(sid * SLC, SLC)])
    for k in range(8):
        ones_v[pl.ds(k * 16, 16)] = jnp.full((16,), 1.0, jnp.float32)

    for r in range(ROWS):
        for c in range(8):
            o = r * 128 + c * 16
            iv = d_v[pl.ds(o, 16)] * N + s_v[pl.ds(o, 16)]
            idx2[r, pl.ds(c * 16, 16)] = iv

    plsc.subcore_barrier()

    handles = []
    for r in range(ROWS):
        handles.append(pltpu.async_copy(ones_v, cnt_sh.at[idx2.at[r]], sem, add=True))
    for h in handles:
        h.wait()

    plsc.subcore_barrier()

    pltpu.sync_copy(cnt_sh.at[pl.ds(sid * SLC, SLC)],
                    out_hbm.at[cid, pl.ds(sid * SLC, SLC)])


def _build_counts(edge_index):
    src = jnp.concatenate([edge_index[0], jnp.zeros((EPAD - E,), jnp.int32)])
    dst = jnp.concatenate([edge_index[1], jnp.zeros((EPAD - E,), jnp.int32)])
    zero = jnp.zeros((EPAD,), jnp.float32)
    counts = pl.kernel(
        _count_body,
        mesh=plsc.VectorSubcoreMesh(core_axis_name="c", subcore_axis_name="s"),
        out_type=jax.ShapeDtypeStruct((NC, EPAD), jnp.float32),
        scratch_types=[
            pltpu.VMEM((EPT,), jnp.int32),
            pltpu.VMEM((EPT,), jnp.int32),
            pltpu.VMEM((ROWS, 128), jnp.int32),
            pltpu.VMEM((128,), jnp.float32),
            pltpu.VMEM_SHARED((EPAD,), jnp.float32),
            pltpu.SemaphoreType.DMA,
        ],
    )(src, dst, zero)
    return counts.reshape(NC, N, N)


def _gnn_body(c2_ref, x_ref, w1r_ref, w1o_ref, b1_ref,
              w2r_ref, w2o_ref, b2_ref, w3_ref, as_ref, ad_ref, b3_ref,
              wres_ref, bres_ref, h_ref):
    f32 = jnp.float32
    row_i = jax.lax.broadcasted_iota(jnp.int32, (N, N), 0)
    col_i = jax.lax.broadcasted_iota(jnp.int32, (N, N), 1)
    corner = ((row_i == 0) & (col_i == 0)).astype(f32)
    C = c2_ref[0] + c2_ref[1] - float(EPAD - E) * corner

    xdot = functools.partial(jnp.dot, preferred_element_type=f32,
                             precision=jax.lax.Precision.HIGHEST)
    dot = functools.partial(jnp.dot, preferred_element_type=f32)
    relu = lambda v: jnp.maximum(v, 0.0)

    x = x_ref[:]
    agg1 = xdot(C, x)
    h1 = relu(dot(agg1, w1r_ref[:]) + dot(x, w1o_ref[:]) + b1_ref[:]
              + dot(x, wres_ref[:]) + bres_ref[:])
    h2 = relu(dot(xdot(C, h1), w2r_ref[:]) + dot(h1, w2o_ref[:]) + b2_ref[:])

    Cp = C + (row_i == col_i).astype(f32)
    mask = Cp > 0.0
    xp = dot(h2, w3_ref[:])
    acc = jnp.zeros((N, HID), f32)
    for hh in range(HEADS):
        xph = xp[:, hh * HID:(hh + 1) * HID]
        al_s = jnp.sum(xph * as_ref[hh, :][None, :], axis=1)
        al_d = jnp.sum(xph * ad_ref[hh, :][None, :], axis=1)
        e = al_d[:, None] + al_s[None, :]
        e = jnp.where(e >= 0.0, e, 0.2 * e)
        m = jnp.max(jnp.where(mask, e, -1e30), axis=1, keepdims=True)
        ee = Cp * jnp.exp(jnp.minimum(e - m, 0.0))
        denom = jnp.sum(ee, axis=1, keepdims=True)
        acc = acc + xdot(ee, xph) / denom
    h_ref[:] = relu(acc * (1.0 / HEADS) + b3_ref[:])


def _fc_body(h0_ref, h1_ref, h2_ref, h3_ref, w0_ref, w1_ref, w2_ref, w3_ref,
             b_ref, o_ref):
    k = pl.program_id(0)

    @pl.when(k == 0)
    def _():
        o_ref[:] = b_ref[:]

    acc = jnp.dot(h0_ref[0], w0_ref[:], preferred_element_type=jnp.float32)
    acc += jnp.dot(h1_ref[0], w1_ref[:], preferred_element_type=jnp.float32)
    acc += jnp.dot(h2_ref[0], w2_ref[:], preferred_element_type=jnp.float32)
    acc += jnp.dot(h3_ref[0], w3_ref[:], preferred_element_type=jnp.float32)
    o_ref[:] += acc


def kernel(x, edge_index, W1_rel, W1_root, b1, W2_rel, W2_root, b2, W3,
           a_src, a_dst, b3, Wres, bres, Wfc, bfc):
    c2 = _build_counts(edge_index)

    h = pl.pallas_call(
        _gnn_body,
        out_shape=jax.ShapeDtypeStruct((N, HID), jnp.float32),
    )(c2, x, W1_rel, W1_root, b1.reshape(1, HID),
      W2_rel, W2_root, b2.reshape(1, HID), W3, a_src, a_dst,
      b3.reshape(1, HID), Wres, bres)

    h3d = h.reshape(N, 1, HID)
    h_specs = [pl.BlockSpec((1, 1, HID), (lambda k, q=q: (q * FCK + k, 0, 0)))
               for q in range(FCS)]
    w_specs = [pl.BlockSpec((HID, OUT), (lambda k, q=q: (q * FCK + k, 0)))
               for q in range(FCS)]
    out = pl.pallas_call(
        _fc_body,
        grid=(FCK,),
        in_specs=h_specs + w_specs + [pl.BlockSpec((1, OUT), lambda k: (0, 0))],
        out_specs=pl.BlockSpec((1, OUT), lambda k: (0, 0)),
        out_shape=jax.ShapeDtypeStruct((1, OUT), jnp.float32),
    )(h3d, h3d, h3d, h3d, Wfc, Wfc, Wfc, Wfc, bfc.reshape(1, OUT))
    return out


# speedup vs baseline: 1.2087x; 1.2087x over previous
import functools

import jax
import jax.numpy as jnp
from jax import lax
from jax.experimental import pallas as pl
from jax.experimental.pallas import tpu as pltpu
from jax.experimental.pallas import tpu_sc as plsc

N = 256
E = 65280
EPAD = 65536
IN = 7
HID = 128
HEADS = 2
OUT = N * 6

NC = 2
NS = 16
EPT = EPAD // (NC * NS)
EREAL = E // (NC * NS)
ROWS = EPT // 128
SLC = EPAD // NS

FCK = 16
FKB = (N * HID) // FCK


def _count_body(edge_hbm, zero_hbm, out_hbm,
                s_v, d_v, idx2, ones_v, cnt_sh, sem):
    cid = lax.axis_index("c")
    sid = lax.axis_index("s")
    tid = cid * NS + sid
    base = tid * EREAL

    pltpu.sync_copy(edge_hbm.at[pl.ds(base, EREAL)], s_v.at[pl.ds(0, EREAL)])
    pltpu.sync_copy(edge_hbm.at[pl.ds(E + base, EREAL)], d_v.at[pl.ds(0, EREAL)])
    pltpu.sync_copy(zero_hbm.at[pl.ds(sid * SLC, SLC)], cnt_sh.at[pl.ds(sid * SLC, SLC)])
    for k in range(8):
        ones_v[pl.ds(k * 16, 16)] = jnp.full((16,), 1.0, jnp.float32)

    lane = lax.iota(jnp.int32, 16)
    for r in range(ROWS):
        for c in range(8):
            o = r * 128 + c * 16
            iv = d_v[pl.ds(o, 16)] * N + s_v[pl.ds(o, 16)]
            if o + 16 > EREAL:
                iv = jnp.where(lane < EREAL - o, iv, 0)
            idx2[r, pl.ds(c * 16, 16)] = iv

    plsc.subcore_barrier()

    handles = []
    for r in range(ROWS):
        handles.append(pltpu.async_copy(ones_v, cnt_sh.at[idx2.at[r]], sem, add=True))
    for h in handles:
        h.wait()

    plsc.subcore_barrier()

    pltpu.sync_copy(cnt_sh.at[pl.ds(sid * SLC, SLC)],
                    out_hbm.at[cid, pl.ds(sid * SLC, SLC)])


def _build_counts(edge_index):
    zero = jnp.zeros((EPAD,), jnp.float32)
    return pl.kernel(
        _count_body,
        mesh=plsc.VectorSubcoreMesh(core_axis_name="c", subcore_axis_name="s"),
        out_type=jax.ShapeDtypeStruct((NC, EPAD), jnp.float32),
        scratch_types=[
            pltpu.VMEM((EPT,), jnp.int32),
            pltpu.VMEM((EPT,), jnp.int32),
            pltpu.VMEM((ROWS, 128), jnp.int32),
            pltpu.VMEM((128,), jnp.float32),
            pltpu.VMEM_SHARED((EPAD,), jnp.float32),
            pltpu.SemaphoreType.DMA,
        ],
    )(edge_index.reshape(2 * E), zero)


def _gnn_body(c2_ref, x_ref, w1r_ref, w1o_ref, b1_ref,
              w2r_ref, w2o_ref, b2_ref, w3_ref, as_ref, ad_ref, b3_ref,
              wres_ref, bres_ref, h_ref):
    f32 = jnp.float32
    row_i = jax.lax.broadcasted_iota(jnp.int32, (N, N), 0)
    col_i = jax.lax.broadcasted_iota(jnp.int32, (N, N), 1)
    corner = ((row_i == 0) & (col_i == 0)).astype(f32)
    C = (c2_ref[0] + c2_ref[1]).reshape(N, N) - float(EPAD - E) * corner

    xdot = functools.partial(jnp.dot, preferred_element_type=f32,
                             precision=jax.lax.Precision.HIGHEST)
    dot = functools.partial(jnp.dot, preferred_element_type=f32)
    relu = lambda v: jnp.maximum(v, 0.0)

    x = x_ref[:]
    agg1 = xdot(C, x)
    h1 = relu(dot(agg1, w1r_ref[:]) + dot(x, w1o_ref[:]) + b1_ref[:]
              + dot(x, wres_ref[:]) + bres_ref[:])
    h2 = relu(dot(xdot(C, h1), w2r_ref[:]) + dot(h1, w2o_ref[:]) + b2_ref[:])

    Cp = C + (row_i == col_i).astype(f32)
    mask = Cp > 0.0
    xp = dot(h2, w3_ref[:])
    acc = jnp.zeros((N, HID), f32)
    for hh in range(HEADS):
        xph = xp[:, hh * HID:(hh + 1) * HID]
        al_s = jnp.sum(xph * as_ref[hh, :][None, :], axis=1)
        al_d = jnp.sum(xph * ad_ref[hh, :][None, :], axis=1)
        e = al_d[:, None] + al_s[None, :]
        e = jnp.where(e >= 0.0, e, 0.2 * e)
        m = jnp.max(jnp.where(mask, e, -1e30), axis=1, keepdims=True)
        ee = Cp * jnp.exp(jnp.minimum(e - m, 0.0))
        denom = jnp.sum(ee, axis=1, keepdims=True)
        acc = acc + xdot(ee, xph) / denom
    h_ref[:] = relu(acc * (1.0 / HEADS) + b3_ref[:])


def _fc_body(flat_ref, w_ref, b_ref, o_ref):
    k = pl.program_id(0)

    @pl.when(k == 0)
    def _():
        o_ref[:] = b_ref[:]

    o_ref[:] += jnp.dot(flat_ref[:], w_ref[:], preferred_element_type=jnp.float32)


def kernel(x, edge_index, W1_rel, W1_root, b1, W2_rel, W2_root, b2, W3,
           a_src, a_dst, b3, Wres, bres, Wfc, bfc):
    c2 = _build_counts(edge_index)

    h = pl.pallas_call(
        _gnn_body,
        out_shape=jax.ShapeDtypeStruct((N, HID), jnp.float32),
    )(c2, x, W1_rel, W1_root, b1.reshape(1, HID),
      W2_rel, W2_root, b2.reshape(1, HID), W3, a_src, a_dst,
      b3.reshape(1, HID), Wres, bres)

    flat = h.reshape(1, N * HID)
    out = pl.pallas_call(
        _fc_body,
        grid=(FCK,),
        in_specs=[
            pl.BlockSpec((1, FKB), lambda k: (0, k)),
            pl.BlockSpec((FKB, OUT), lambda k: (k, 0)),
            pl.BlockSpec((1, OUT), lambda k: (0, 0)),
        ],
        out_specs=pl.BlockSpec((1, OUT), lambda k: (0, 0)),
        out_shape=jax.ShapeDtypeStruct((1, OUT), jnp.float32),
    )(flat, Wfc, bfc.reshape(1, OUT))
    return out


# speedup vs baseline: 1.2658x; 1.0473x over previous
import functools

import jax
import jax.numpy as jnp
from jax import lax
from jax.experimental import pallas as pl
from jax.experimental.pallas import tpu as pltpu
from jax.experimental.pallas import tpu_sc as plsc

N = 256
E = 65280
EPAD = 65536
IN = 7
HID = 128
HEADS = 2
OUT = N * 6

NC = 2
NS = 16
EPT = EPAD // (NC * NS)
EREAL = E // (NC * NS)
ROWS = EPT // 128
SLC = EPAD // NS

FCK = 16
FKB = (N * HID) // FCK


def _count_body(edge_hbm, zero_hbm, out_hbm,
                s_v, d_v, idx2, ones_v, cnt_sh, sem):
    cid = lax.axis_index("c")
    sid = lax.axis_index("s")
    tid = cid * NS + sid
    base = tid * EREAL

    pltpu.sync_copy(edge_hbm.at[pl.ds(base, EREAL)], s_v.at[pl.ds(0, EREAL)])
    pltpu.sync_copy(edge_hbm.at[pl.ds(E + base, EREAL)], d_v.at[pl.ds(0, EREAL)])
    pltpu.sync_copy(zero_hbm.at[pl.ds(sid * SLC, SLC)], cnt_sh.at[pl.ds(sid * SLC, SLC)])
    for k in range(8):
        ones_v[pl.ds(k * 16, 16)] = jnp.full((16,), 1.0, jnp.float32)

    lane = lax.iota(jnp.int32, 16)
    for r in range(ROWS):
        for c in range(8):
            o = r * 128 + c * 16
            iv = d_v[pl.ds(o, 16)] * N + s_v[pl.ds(o, 16)]
            if o + 16 > EREAL:
                iv = jnp.where(lane < EREAL - o, iv, 0)
            idx2[r, pl.ds(c * 16, 16)] = iv

    plsc.subcore_barrier()

    handles = []
    for r in range(ROWS):
        handles.append(pltpu.async_copy(ones_v, cnt_sh.at[idx2.at[r]], sem, add=True))
    for h in handles:
        h.wait()

    plsc.subcore_barrier()

    pltpu.sync_copy(cnt_sh.at[pl.ds(sid * SLC, SLC)],
                    out_hbm.at[cid, pl.ds(sid * SLC, SLC)])


def _build_counts(edge_index):
    zero = jnp.zeros((EPAD,), jnp.float32)
    return pl.kernel(
        _count_body,
        mesh=plsc.VectorSubcoreMesh(core_axis_name="c", subcore_axis_name="s"),
        out_type=jax.ShapeDtypeStruct((NC, EPAD), jnp.float32),
        scratch_types=[
            pltpu.VMEM((EPT,), jnp.int32),
            pltpu.VMEM((EPT,), jnp.int32),
            pltpu.VMEM((ROWS, 128), jnp.int32),
            pltpu.VMEM((128,), jnp.float32),
            pltpu.VMEM_SHARED((EPAD,), jnp.float32),
            pltpu.SemaphoreType.DMA,
        ],
    )(edge_index.reshape(2 * E), zero)


def _gnn_fc_body(c2_ref, x_ref, w1r_ref, w1o_ref, b1_ref,
                 w2r_ref, w2o_ref, b2_ref, w3_ref, as_ref, ad_ref, b3_ref,
                 wres_ref, bres_ref, wfc_ref, bfc_ref, o_ref, h_s):
    k = pl.program_id(0)

    @pl.when(k == 0)
    def _():
        _gnn_compute(c2_ref, x_ref, w1r_ref, w1o_ref, b1_ref,
                     w2r_ref, w2o_ref, b2_ref, w3_ref, as_ref, ad_ref, b3_ref,
                     wres_ref, bres_ref, h_s)
        o_ref[:] = bfc_ref[:]

    flat_blk = h_s[pl.ds(k * (FKB // HID), FKB // HID), :].reshape(1, FKB)
    o_ref[:] += jnp.dot(flat_blk, wfc_ref[:], preferred_element_type=jnp.float32)


def _gnn_compute(c2_ref, x_ref, w1r_ref, w1o_ref, b1_ref,
                 w2r_ref, w2o_ref, b2_ref, w3_ref, as_ref, ad_ref, b3_ref,
                 wres_ref, bres_ref, h_ref):
    f32 = jnp.float32
    row_i = jax.lax.broadcasted_iota(jnp.int32, (N, N), 0)
    col_i = jax.lax.broadcasted_iota(jnp.int32, (N, N), 1)
    corner = ((row_i == 0) & (col_i == 0)).astype(f32)
    C = (c2_ref[0] + c2_ref[1]).reshape(N, N) - float(EPAD - E) * corner

    xdot = functools.partial(jnp.dot, preferred_element_type=f32,
                             precision=jax.lax.Precision.HIGHEST)
    dot = functools.partial(jnp.dot, preferred_element_type=f32)
    relu = lambda v: jnp.maximum(v, 0.0)

    x = x_ref[:]
    agg1 = xdot(C, x)
    h1 = relu(dot(agg1, w1r_ref[:]) + dot(x, w1o_ref[:]) + b1_ref[:]
              + dot(x, wres_ref[:]) + bres_ref[:])
    h2 = relu(dot(xdot(C, h1), w2r_ref[:]) + dot(h1, w2o_ref[:]) + b2_ref[:])

    Cp = C + (row_i == col_i).astype(f32)
    mask = Cp > 0.0
    xp = dot(h2, w3_ref[:])
    acc = jnp.zeros((N, HID), f32)
    for hh in range(HEADS):
        xph = xp[:, hh * HID:(hh + 1) * HID]
        al_s = jnp.sum(xph * as_ref[hh, :][None, :], axis=1)
        al_d = jnp.sum(xph * ad_ref[hh, :][None, :], axis=1)
        e = al_d[:, None] + al_s[None, :]
        e = jnp.where(e >= 0.0, e, 0.2 * e)
        m = jnp.max(jnp.where(mask, e, -1e30), axis=1, keepdims=True)
        ee = Cp * jnp.exp(jnp.minimum(e - m, 0.0))
        denom = jnp.sum(ee, axis=1, keepdims=True)
        acc = acc + xdot(ee, xph) / denom
    h_ref[:] = relu(acc * (1.0 / HEADS) + b3_ref[:])


def kernel(x, edge_index, W1_rel, W1_root, b1, W2_rel, W2_root, b2, W3,
           a_src, a_dst, b3, Wres, bres, Wfc, bfc):
    c2 = _build_counts(edge_index)

    const = lambda k: (0, 0)
    out = pl.pallas_call(
        _gnn_fc_body,
        grid=(FCK,),
        in_specs=[
            pl.BlockSpec((NC, EPAD), const),
            pl.BlockSpec((N, IN), const),
            pl.BlockSpec((IN, HID), const),
            pl.BlockSpec((IN, HID), const),
            pl.BlockSpec((1, HID), const),
            pl.BlockSpec((HID, HID), const),
            pl.BlockSpec((HID, HID), const),
            pl.BlockSpec((1, HID), const),
            pl.BlockSpec((HID, HEADS * HID), const),
            pl.BlockSpec((HEADS, HID), const),
            pl.BlockSpec((HEADS, HID), const),
            pl.BlockSpec((1, HID), const),
            pl.BlockSpec((IN, HID), const),
            pl.BlockSpec((1, HID), const),
            pl.BlockSpec((FKB, OUT), lambda k: (k, 0)),
            pl.BlockSpec((1, OUT), const),
        ],
        out_specs=pl.BlockSpec((1, OUT), const),
        out_shape=jax.ShapeDtypeStruct((1, OUT), jnp.float32),
        scratch_shapes=[pltpu.VMEM((N, HID), jnp.float32)],
    )(c2, x, W1_rel, W1_root, b1.reshape(1, HID),
      W2_rel, W2_root, b2.reshape(1, HID), W3, a_src, a_dst,
      b3.reshape(1, HID), Wres, bres.reshape(1, HID), Wfc, bfc.reshape(1, OUT))
    return out
